# Initial kernel scaffold; baseline (speedup 1.0000x reference)
import jax
import jax.numpy as jnp
from jax.experimental import pallas as pl


def kernel(O, A, D, seq_len, obs_mem, act_mem, dne_mem, obs_buf, act_buf, dne_buf, buf_indexes, mem_index):
    raise NotImplementedError("write your pallas kernel here")


# speedup vs baseline: 7.4109x; 7.4109x over previous
import jax
import jax.numpy as jnp
from jax import lax
from jax.experimental import pallas as pl
from jax.experimental.pallas import tpu as pltpu, tpu_sc as plsc
import functools

_NC = 2
_NS = 16
_NW = _NC * _NS
_L = 16


def _build(B, T, M, Do):
    mpt = M // _NW
    mesh = plsc.VectorSubcoreMesh(core_axis_name="c", subcore_axis_name="s")

    @functools.partial(
        pl.kernel,
        out_type=jax.ShapeDtypeStruct((M, T, Do), jnp.float32),
        mesh=mesh,
        scratch_types=[
            pltpu.VMEM((B,), jnp.int32),
            pltpu.VMEM((B,), jnp.int32),
            pltpu.VMEM((B,), jnp.int32),
            pltpu.VMEM((_L,), jnp.int32),
            pltpu.VMEM((T, Do), jnp.float32),
        ],
        compiler_params=pltpu.CompilerParams(needs_layout_passes=False),
    )
    def k(o_hbm, d_hbm, bi_hbm, obs_buf_hbm, mem_in_hbm, meta_hbm, out_hbm,
          d_v, bi_v, cs_v, meta_v, blk):
        wid = lax.axis_index("s") * _NC + lax.axis_index("c")
        pltpu.sync_copy(d_hbm, d_v)
        pltpu.sync_copy(bi_hbm, bi_v)
        pltpu.sync_copy(meta_hbm, meta_v)

        meta = meta_v[...]
        base = meta[0] % M
        seq_len = meta[1]

        def cs_step(i, carry):
            dvec = d_v[pl.ds(i * _L, _L)]
            bvec = bi_v[pl.ds(i * _L, _L)]
            fired = dvec > 0
            done = jnp.logical_and(
                jnp.logical_or(fired, bvec == T - 1),
                jnp.logical_not(jnp.logical_and(fired, bvec < seq_len)))
            cs = plsc.cumsum(done.astype(jnp.int32)) + carry
            cs_v[pl.ds(i * _L, _L)] = cs
            return jnp.max(cs)

        lax.fori_loop(0, B // _L, cs_step, jnp.int32(0))
        total = cs_v[pl.ds(B - _L, _L)][_L - 1]

        lanes = lax.iota(jnp.int32, _L)
        for g in range(mpt // _L):
            mvec = wid * mpt + g * _L + lanes
            rvec = mvec - base
            tvec = rvec + 1
            src = jnp.zeros((_L,), jnp.int32)
            for sh in (1024, 512, 256, 128, 64, 32, 16, 8, 4, 2, 1):
                cand = src + sh
                c = plsc.load_gather(cs_v, [cand - 1])
                src = jnp.where(c < tvec, cand, src)
            valid = jnp.logical_and(rvec >= 0, rvec < total).astype(jnp.int32)
            cur = plsc.load_gather(bi_v, [src])

            for ln in range(_L):
                m = wid * mpt + g * _L + ln
                src_j = src[ln]
                cur_j = cur[ln]
                valid_j = valid[ln] != 0

                @pl.when(valid_j)
                def _():
                    pltpu.sync_copy(obs_buf_hbm.at[src_j], blk)
                    pltpu.sync_copy(o_hbm.at[src_j], blk.at[cur_j])

                @pl.when(jnp.logical_not(valid_j))
                def _():
                    pltpu.sync_copy(mem_in_hbm.at[m], blk)

                pltpu.sync_copy(blk, out_hbm.at[m])

    return k


def kernel(O, A, D, seq_len, obs_mem, act_mem, dne_mem, obs_buf, act_buf,
           dne_buf, buf_indexes, mem_index):
    B, T, Do = obs_buf.shape
    M = obs_mem.shape[0]
    d_flat = D.reshape(B).astype(jnp.int32)
    bi = buf_indexes.astype(jnp.int32)
    meta = (jnp.zeros((_L,), jnp.int32)
            .at[0].set(mem_index[0].astype(jnp.int32))
            .at[1].set(jnp.asarray(seq_len, jnp.int32)))
    k = _build(B, T, M, Do)
    return k(O, d_flat, bi, obs_buf, obs_mem, meta)
